# Initial kernel scaffold; baseline (speedup 1.0000x reference)
import jax
import jax.numpy as jnp
from jax.experimental import pallas as pl


def kernel(x, edge_index, edge_weight, batch, W1, b1, g1, be1, W2, b2, g2, be2, W3, b3, g3, be3, W4, b4, g4, be4, W5, b5, g5, be5, Wfc1, bfc1, Wfc2, bfc2):
    raise NotImplementedError("write your pallas kernel here")


# speedup vs baseline: 6.4742x; 6.4742x over previous
import functools

import jax
import jax.numpy as jnp
from jax import lax
from jax.experimental import pallas as pl
from jax.experimental.pallas import tpu as pltpu
from jax.experimental.pallas import tpu_sc as plsc

_N = 10000
_E = 320000
_G = 64
_NC = 2
_NS = 16
_NW = _NC * _NS
_CH = 128
_ETOT = _E + _N
_NCHUNK = -(-_ETOT // (_NW * _CH))
_EPAD = _NW * _CH * _NCHUNK
_STRIPE = 632
_NPAD = _NS * _STRIPE

_SC_PARAMS = pltpu.CompilerParams(use_tc_tiling_on_sc=False)


@functools.lru_cache(maxsize=None)
def _mesh():
    return plsc.VectorSubcoreMesh(core_axis_name="c", subcore_axis_name="s")


def _wid_base():
    c = lax.axis_index("c")
    s = lax.axis_index("s")
    return c, s, (c * _NS + s) * _NCHUNK


@functools.lru_cache(maxsize=None)
def _make_degree():
  @functools.partial(
    pl.kernel,
    out_type=jax.ShapeDtypeStruct((_NC, _NPAD, 16), jnp.float32),
    mesh=_mesh(),
    compiler_params=_SC_PARAMS,
    scratch_types=[
        pltpu.VMEM((_CH,), jnp.int32),
        pltpu.VMEM((_CH,), jnp.float32),
        pltpu.VMEM((_CH, 16), jnp.float32),
        pltpu.VMEM_SHARED((_NPAD, 16), jnp.float32),
    ],
  )
  def _sc_degree(col_hbm, ew_hbm, zeros_hbm, out_hbm, colb, ewb, rows, acc):
    c, s, base = _wid_base()
    pltpu.sync_copy(zeros_hbm.at[pl.ds(s * _STRIPE, _STRIPE)],
                    acc.at[pl.ds(s * _STRIPE, _STRIPE)])
    plsc.subcore_barrier()

    def chunk_body(j, carry):
        off = (base + j) * _CH
        pltpu.sync_copy(col_hbm.at[pl.ds(off, _CH)], colb)
        pltpu.sync_copy(ew_hbm.at[pl.ds(off, _CH)], ewb)

        def splat(jj, ecarry):
            wv = ewb[pl.ds(jj * 16, 16)]
            for i in range(16):
                rows[jj * 16 + i, pl.ds(0, 16)] = jnp.full((16,), wv[i], jnp.float32)
            return ecarry

        lax.fori_loop(0, _CH // 16, splat, 0)
        pltpu.sync_copy(rows, acc.at[colb], add=True)
        return carry

    lax.fori_loop(0, _NCHUNK, chunk_body, 0)
    plsc.subcore_barrier()
    pltpu.sync_copy(acc.at[pl.ds(s * _STRIPE, _STRIPE)],
                    out_hbm.at[c, pl.ds(s * _STRIPE, _STRIPE)])

  return _sc_degree


@functools.lru_cache(maxsize=None)
def _make_norm():
  @functools.partial(
    pl.kernel,
    out_type=jax.ShapeDtypeStruct((_EPAD, 16), jnp.float32),
    mesh=_mesh(),
    compiler_params=_SC_PARAMS,
    scratch_types=[
        pltpu.VMEM((_CH,), jnp.int32),
        pltpu.VMEM((_CH,), jnp.int32),
        pltpu.VMEM((_CH,), jnp.float32),
        pltpu.VMEM((_CH, 16), jnp.float32),
        pltpu.VMEM((_CH, 16), jnp.float32),
    ],
  )
  def _sc_norm(row_hbm, col_hbm, ew_hbm, dis16_hbm, out_hbm,
             rowb, colb, ewb, disr, disc):
    _, _, base = _wid_base()

    def chunk_body(j, carry):
        off = (base + j) * _CH
        pltpu.sync_copy(row_hbm.at[pl.ds(off, _CH)], rowb)
        pltpu.sync_copy(col_hbm.at[pl.ds(off, _CH)], colb)
        pltpu.sync_copy(ew_hbm.at[pl.ds(off, _CH)], ewb)
        pltpu.sync_copy(dis16_hbm.at[rowb], disr)
        pltpu.sync_copy(dis16_hbm.at[colb], disc)

        def emul(jj, ecarry):
            wv = ewb[pl.ds(jj * 16, 16)]
            for i in range(16):
                e = jj * 16 + i
                sl = pl.ds(0, 16)
                disr[e, sl] = (disr[e, sl] * wv[i]) * disc[e, sl]
            return ecarry

        lax.fori_loop(0, _CH // 16, emul, 0)
        pltpu.sync_copy(disr, out_hbm.at[pl.ds(off, _CH)])
        return carry

    lax.fori_loop(0, _NCHUNK, chunk_body, 0)

  return _sc_norm


@functools.lru_cache(maxsize=None)
def _make_propagate(D):
    @functools.partial(
        pl.kernel,
        out_type=jax.ShapeDtypeStruct((_NC, _NPAD, D), jnp.float32),
        mesh=_mesh(),
        compiler_params=_SC_PARAMS,
        scratch_types=[
            pltpu.VMEM((_CH,), jnp.int32),
            pltpu.VMEM((_CH,), jnp.int32),
            pltpu.VMEM((_CH, 16), jnp.float32),
            pltpu.VMEM((_CH, D), jnp.float32),
            pltpu.VMEM_SHARED((_NPAD, D), jnp.float32),
        ],
    )
    def prop(src_hbm, row_hbm, col_hbm, norm_hbm, zeros_hbm, out_hbm,
             rowb, colb, nb, rows, acc):
        c, s, base = _wid_base()
        pltpu.sync_copy(zeros_hbm.at[pl.ds(s * _STRIPE, _STRIPE)],
                        acc.at[pl.ds(s * _STRIPE, _STRIPE)])
        plsc.subcore_barrier()

        def chunk_body(j, carry):
            off = (base + j) * _CH
            pltpu.sync_copy(row_hbm.at[pl.ds(off, _CH)], rowb)
            pltpu.sync_copy(col_hbm.at[pl.ds(off, _CH)], colb)
            pltpu.sync_copy(norm_hbm.at[pl.ds(off, _CH)], nb)
            pltpu.sync_copy(src_hbm.at[rowb], rows)

            def escale(e, ecarry):
                nv = nb[e, pl.ds(0, 16)]
                for k in range(D // 16):
                    sl = pl.ds(k * 16, 16)
                    rows[e, sl] = rows[e, sl] * nv
                return ecarry

            lax.fori_loop(0, _CH, escale, 0)
            pltpu.sync_copy(rows, acc.at[colb], add=True)
            return carry

        lax.fori_loop(0, _NCHUNK, chunk_body, 0)
        plsc.subcore_barrier()
        pltpu.sync_copy(acc.at[pl.ds(s * _STRIPE, _STRIPE)],
                        out_hbm.at[c, pl.ds(s * _STRIPE, _STRIPE)])

    return prop


def _bn_in_kernel(u, g, be):
    m = jnp.mean(u, axis=0, keepdims=True)
    v = jnp.mean((u - m) ** 2, axis=0, keepdims=True)
    return (u - m) * lax.rsqrt(v + 1e-5) * g + be


def _tc_first(deg_parts, x, W1):
    def body(dp_ref, x_ref, w_ref, xw_ref, dis_ref):
        dp = dp_ref[...]
        deg = dp[0, :_N] + dp[1, :_N]
        dis_ref[...] = jnp.where(deg > 0, lax.rsqrt(deg), 0.0)
        xw_ref[...] = jnp.dot(x_ref[...], w_ref[...],
                              preferred_element_type=jnp.float32)

    return pl.pallas_call(
        body,
        out_shape=[jax.ShapeDtypeStruct((_N, 32), jnp.float32),
                   jax.ShapeDtypeStruct((_N, 16), jnp.float32)],
    )(deg_parts, x, W1)


def _tc_mid(parts, b, g, be, mode, *Ws):
    def body(*refs):
        p_ref, b_ref, g_ref, be_ref = refs[:4]
        w_refs = refs[4:4 + len(Ws)]
        o_refs = refs[4 + len(Ws):]
        p = p_ref[...]
        t = (p[0, :_N] + p[1, :_N]) + b_ref[...]
        if mode == "relu_bn":
            h = _bn_in_kernel(jax.nn.relu(t), g_ref[...], be_ref[...])
        else:
            h = jax.nn.relu(_bn_in_kernel(t, g_ref[...], be_ref[...]))
        for w_ref, o_ref in zip(w_refs, o_refs):
            o_ref[...] = jnp.dot(h, w_ref[...],
                                 preferred_element_type=jnp.float32)

    return pl.pallas_call(
        body,
        out_shape=[jax.ShapeDtypeStruct((_N, W.shape[1]), jnp.float32)
                   for W in Ws],
    )(parts, b, g, be, *Ws)


def _tc_final(parts_a, parts_b, b5, g5, be5, batch2, Wfc1, bfc1, Wfc2, bfc2):
    def body(pa_ref, pb_ref, b_ref, g_ref, be_ref, bt_ref,
             wf1_ref, bf1_ref, wf2_ref, bf2_ref, o_ref):
        pa = pa_ref[...]
        pb = pb_ref[...]
        t = jnp.concatenate([pa[0, :_N] + pa[1, :_N],
                             pb[0, :_N] + pb[1, :_N]], axis=1) + b_ref[...]
        h = _bn_in_kernel(t, g_ref[...], be_ref[...])
        oh = (bt_ref[...] == lax.broadcasted_iota(jnp.int32, (1, _G), 1))
        oh = oh.astype(jnp.float32)
        pooled = lax.dot_general(oh, h, (((0,), (0,)), ((), ())),
                                 preferred_element_type=jnp.float32,
                                 precision=lax.Precision.HIGHEST)
        pp = jax.nn.relu(pooled)
        pp = jax.nn.relu(
            jnp.dot(pp, wf1_ref[...], preferred_element_type=jnp.float32)
            + bf1_ref[...])
        o_ref[...] = jnp.dot(pp, wf2_ref[...],
                             preferred_element_type=jnp.float32) + bf2_ref[...]

    return pl.pallas_call(
        body,
        out_shape=jax.ShapeDtypeStruct((_G, 1), jnp.float32),
    )(parts_a, parts_b, b5, g5, be5, batch2, Wfc1, bfc1, Wfc2, bfc2)


def kernel(x, edge_index, edge_weight, batch,
           W1, b1, g1, be1, W2, b2, g2, be2, W3, b3, g3, be3,
           W4, b4, g4, be4, W5, b5, g5, be5, Wfc1, bfc1, Wfc2, bfc2):
    loop = jnp.arange(_N, dtype=edge_index.dtype)
    pad = _EPAD - _ETOT
    padidx = (jnp.arange(pad, dtype=jnp.int32) * 37) % _N
    row = jnp.concatenate([edge_index[0], loop, padidx])
    col = jnp.concatenate([edge_index[1], loop, padidx])
    ew = jnp.concatenate([edge_weight, jnp.ones((_N,), edge_weight.dtype),
                          jnp.zeros((pad,), edge_weight.dtype)])

    r2 = lambda a: a.reshape(1, -1)
    z16 = jnp.zeros((_NPAD, 16), jnp.float32)
    z32 = jnp.zeros((_NPAD, 32), jnp.float32)
    z64 = jnp.zeros((_NPAD, 64), jnp.float32)
    z128 = jnp.zeros((_NPAD, 128), jnp.float32)

    deg_parts = _make_degree()(col, ew, z16)
    xw1, dis16 = _tc_first(deg_parts, x, W1)
    norm = _make_norm()(row, col, ew, dis16)

    p1 = _make_propagate(32)(xw1, row, col, norm, z32)
    (xw2,) = _tc_mid(p1, r2(b1), r2(g1), r2(be1), "relu_bn", W2)

    p2 = _make_propagate(64)(xw2, row, col, norm, z64)
    (xw3,) = _tc_mid(p2, r2(b2), r2(g2), r2(be2), "relu_bn", W3)

    p3 = _make_propagate(128)(xw3, row, col, norm, z128)
    (xw4,) = _tc_mid(p3, r2(b3), r2(g3), r2(be3), "relu_bn", W4)

    p4 = _make_propagate(128)(xw4, row, col, norm, z128)
    xw5a, xw5b = _tc_mid(p4, r2(b4), r2(g4), r2(be4), "bn_relu",
                         W5[:, :128], W5[:, 128:])

    p5a = _make_propagate(128)(xw5a, row, col, norm, z128)
    p5b = _make_propagate(128)(xw5b, row, col, norm, z128)
    out = _tc_final(p5a, p5b, r2(b5), r2(g5), r2(be5),
                    batch.reshape(_N, 1), Wfc1, r2(bfc1), Wfc2, r2(bfc2))
    return out.reshape(-1)


# speedup vs baseline: 10.1136x; 1.5621x over previous
import functools

import jax
import jax.numpy as jnp
from jax import lax
from jax.experimental import pallas as pl
from jax.experimental.pallas import tpu as pltpu
from jax.experimental.pallas import tpu_sc as plsc

_N = 10000
_E = 320000
_G = 64
_NC = 2
_NS = 16
_NW = _NC * _NS
_CH = 128
_ETOT = _E + _N
_NCHUNK = 84
_EPAD = _NW * _CH * _NCHUNK
_EPAD2 = _EPAD + 2 * 3 * _CH
_KBY = {16: 3, 32: 3, 64: 3, 128: 1}
_TOTCH = _EPAD2 // _CH
_STRIPE = 632
_NPAD = _NS * _STRIPE

_SC_PARAMS = pltpu.CompilerParams(use_tc_tiling_on_sc=False)


@functools.lru_cache(maxsize=None)
def _mesh():
    return plsc.VectorSubcoreMesh(core_axis_name="c", subcore_axis_name="s")


def _wid_base():
    c = lax.axis_index("c")
    s = lax.axis_index("s")
    return c, s, (c * _NS + s) * _NCHUNK


@functools.lru_cache(maxsize=None)
def _make_degree():
  @functools.partial(
    pl.kernel,
    out_type=jax.ShapeDtypeStruct((_NC, _NPAD, 16), jnp.float32),
    mesh=_mesh(),
    compiler_params=_SC_PARAMS,
    scratch_types=[
        pltpu.VMEM((_CH,), jnp.int32),
        pltpu.VMEM((_CH,), jnp.float32),
        pltpu.VMEM((_CH, 16), jnp.float32),
        pltpu.VMEM_SHARED((_NPAD, 16), jnp.float32),
    ],
  )
  def _sc_degree(col_hbm, ew_hbm, zeros_hbm, out_hbm, colb, ewb, rows, acc):
    c, s, base = _wid_base()
    pltpu.sync_copy(zeros_hbm.at[pl.ds(s * _STRIPE, _STRIPE)],
                    acc.at[pl.ds(s * _STRIPE, _STRIPE)])
    plsc.subcore_barrier()

    def chunk_body(j, carry):
        off = (base + j) * _CH
        pltpu.sync_copy(col_hbm.at[pl.ds(off, _CH)], colb)
        pltpu.sync_copy(ew_hbm.at[pl.ds(off, _CH)], ewb)

        def splat(jj, ecarry):
            wv = ewb[pl.ds(jj * 16, 16)]
            for i in range(16):
                rows[jj * 16 + i, pl.ds(0, 16)] = jnp.full((16,), wv[i], jnp.float32)
            return ecarry

        lax.fori_loop(0, _CH // 16, splat, 0)
        pltpu.sync_copy(rows, acc.at[colb], add=True)
        return carry

    lax.fori_loop(0, _NCHUNK, chunk_body, 0)
    plsc.subcore_barrier()
    pltpu.sync_copy(acc.at[pl.ds(s * _STRIPE, _STRIPE)],
                    out_hbm.at[c, pl.ds(s * _STRIPE, _STRIPE)])

  return _sc_degree


@functools.lru_cache(maxsize=None)
def _make_norm():
  @functools.partial(
    pl.kernel,
    out_type=jax.ShapeDtypeStruct((_EPAD2, 16), jnp.float32),
    mesh=_mesh(),
    compiler_params=_SC_PARAMS,
    scratch_types=[
        pltpu.VMEM((_CH,), jnp.int32),
        pltpu.VMEM((_CH,), jnp.int32),
        pltpu.VMEM((_CH,), jnp.float32),
        pltpu.VMEM((_CH, 16), jnp.float32),
        pltpu.VMEM((_CH, 16), jnp.float32),
    ],
  )
  def _sc_norm(row_hbm, col_hbm, ew_hbm, dis16_hbm, out_hbm,
             rowb, colb, ewb, disr, disc):
    _, _, base = _wid_base()

    def chunk_body(j, carry):
        off = (base + j) * _CH
        pltpu.sync_copy(row_hbm.at[pl.ds(off, _CH)], rowb)
        pltpu.sync_copy(col_hbm.at[pl.ds(off, _CH)], colb)
        pltpu.sync_copy(ew_hbm.at[pl.ds(off, _CH)], ewb)
        pltpu.sync_copy(dis16_hbm.at[rowb], disr)
        pltpu.sync_copy(dis16_hbm.at[colb], disc)

        def emul(jj, ecarry):
            wv = ewb[pl.ds(jj * 16, 16)]
            for i in range(16):
                e = jj * 16 + i
                sl = pl.ds(0, 16)
                disr[e, sl] = (disr[e, sl] * wv[i]) * disc[e, sl]
            return ecarry

        lax.fori_loop(0, _CH // 16, emul, 0)
        pltpu.sync_copy(disr, out_hbm.at[pl.ds(off, _CH)])
        return carry

    lax.fori_loop(0, _NCHUNK, chunk_body, 0)

  return _sc_norm


@functools.lru_cache(maxsize=None)
def _make_propagate(D):
    _K = _KBY[D]
    _MAC = _NCHUNK // _K

    @functools.partial(
        pl.kernel,
        out_type=jax.ShapeDtypeStruct((_NC, _NPAD, D), jnp.float32),
        mesh=_mesh(),
        compiler_params=_SC_PARAMS,
        scratch_types=[
            pltpu.VMEM((_K, 2, _CH), jnp.int32),
            pltpu.VMEM((_K, 2, _CH), jnp.int32),
            pltpu.VMEM((_K * _CH, 16), jnp.float32),
            pltpu.VMEM((_K * _CH, 16), jnp.float32),
            pltpu.VMEM((_K * _CH, D), jnp.float32),
            pltpu.VMEM((_K * _CH, D), jnp.float32),
            pltpu.VMEM_SHARED((_NPAD, D), jnp.float32),
            pltpu.SemaphoreType.DMA,
            pltpu.SemaphoreType.DMA,
            pltpu.SemaphoreType.DMA,
            pltpu.SemaphoreType.DMA,
        ],
    )
    def prop(src_hbm, rc_hbm, norm_hbm, zeros_hbm, out_hbm,
             rc0, rc1, nb0, nb1, rows0, rows1, acc, si0, si1, sg0, sg1):
        c, s, base = _wid_base()
        rc = (rc0, rc1)
        nb = (nb0, nb1)
        rows = (rows0, rows1)
        si = (si0, si1)
        sg = (sg0, sg1)

        pltpu.sync_copy(zeros_hbm.at[pl.ds(s * _STRIPE, _STRIPE)],
                        acc.at[pl.ds(s * _STRIPE, _STRIPE)])
        plsc.subcore_barrier()

        def issue_idx(m, b):
            ch0 = base + m * _K
            pltpu.async_copy(rc_hbm.at[pl.ds(ch0, _K)], rc[b], si[b])
            pltpu.async_copy(norm_hbm.at[pl.ds(ch0 * _CH, _K * _CH)],
                             nb[b], si[b])

        def wait_idx(b):
            pltpu.make_async_copy(rc_hbm.at[pl.ds(0, _K)], rc[b], si[b]).wait()
            pltpu.make_async_copy(norm_hbm.at[pl.ds(0, _K * _CH)],
                                  nb[b], si[b]).wait()

        def issue_gather(b):
            for t in range(_K):
                pltpu.async_copy(src_hbm.at[rc[b].at[t, 0]],
                                 rows[b].at[pl.ds(t * _CH, _CH)], sg[b])

        def wait_gather(b):
            for t in range(_K):
                pltpu.make_async_copy(src_hbm.at[rc[b].at[t, 0]],
                                      rows[b].at[pl.ds(t * _CH, _CH)],
                                      sg[b]).wait()

        def macro_step(m, cur):
            nxt = 1 - cur
            wait_gather(cur)
            wait_idx(nxt)
            issue_gather(nxt)

            def escale(e, ecarry):
                nv = nb[cur][e, pl.ds(0, 16)]
                for k in range(D // 16):
                    sl = pl.ds(k * 16, 16)
                    rows[cur][e, sl] = rows[cur][e, sl] * nv
                return ecarry

            lax.fori_loop(0, _K * _CH, escale, 0)
            for t in range(_K):
                pltpu.sync_copy(rows[cur].at[pl.ds(t * _CH, _CH)],
                                acc.at[rc[cur].at[t, 1]], add=True)
            issue_idx(m + 2, cur)

        pltpu.sync_copy(rc_hbm.at[pl.ds(base, _K)], rc0)
        pltpu.sync_copy(norm_hbm.at[pl.ds(base * _CH, _K * _CH)], nb0)
        issue_gather(0)
        issue_idx(1, 1)

        def pair_body(mm, carry):
            macro_step(2 * mm, 0)
            macro_step(2 * mm + 1, 1)
            return carry

        lax.fori_loop(0, _MAC // 2, pair_body, 0)
        wait_gather(0)
        wait_idx(1)

        plsc.subcore_barrier()
        pltpu.sync_copy(acc.at[pl.ds(s * _STRIPE, _STRIPE)],
                        out_hbm.at[c, pl.ds(s * _STRIPE, _STRIPE)])

    return prop


def _bn_in_kernel(u, g, be):
    m = jnp.mean(u, axis=0, keepdims=True)
    v = jnp.mean((u - m) ** 2, axis=0, keepdims=True)
    return (u - m) * lax.rsqrt(v + 1e-5) * g + be


def _tc_first(deg_parts, x, W1):
    def body(dp_ref, x_ref, w_ref, xw_ref, dis_ref):
        dp = dp_ref[...]
        deg = dp[0, :_N] + dp[1, :_N]
        dis_ref[...] = jnp.where(deg > 0, lax.rsqrt(deg), 0.0)
        xw_ref[...] = jnp.dot(x_ref[...], w_ref[...],
                              preferred_element_type=jnp.float32)

    return pl.pallas_call(
        body,
        out_shape=[jax.ShapeDtypeStruct((_N, 32), jnp.float32),
                   jax.ShapeDtypeStruct((_N, 16), jnp.float32)],
    )(deg_parts, x, W1)


def _tc_mid(parts, b, g, be, mode, *Ws):
    def body(*refs):
        p_ref, b_ref, g_ref, be_ref = refs[:4]
        w_refs = refs[4:4 + len(Ws)]
        o_refs = refs[4 + len(Ws):]
        p = p_ref[...]
        t = (p[0, :_N] + p[1, :_N]) + b_ref[...]
        if mode == "relu_bn":
            h = _bn_in_kernel(jax.nn.relu(t), g_ref[...], be_ref[...])
        else:
            h = jax.nn.relu(_bn_in_kernel(t, g_ref[...], be_ref[...]))
        for w_ref, o_ref in zip(w_refs, o_refs):
            o_ref[...] = jnp.dot(h, w_ref[...],
                                 preferred_element_type=jnp.float32)

    return pl.pallas_call(
        body,
        out_shape=[jax.ShapeDtypeStruct((_N, W.shape[1]), jnp.float32)
                   for W in Ws],
    )(parts, b, g, be, *Ws)


def _tc_final(parts_a, parts_b, b5, g5, be5, batch2, Wfc1, bfc1, Wfc2, bfc2):
    def body(pa_ref, pb_ref, b_ref, g_ref, be_ref, bt_ref,
             wf1_ref, bf1_ref, wf2_ref, bf2_ref, o_ref):
        pa = pa_ref[...]
        pb = pb_ref[...]
        t = jnp.concatenate([pa[0, :_N] + pa[1, :_N],
                             pb[0, :_N] + pb[1, :_N]], axis=1) + b_ref[...]
        h = _bn_in_kernel(t, g_ref[...], be_ref[...])
        oh = (bt_ref[...] == lax.broadcasted_iota(jnp.int32, (1, _G), 1))
        oh = oh.astype(jnp.float32)
        pooled = lax.dot_general(oh, h, (((0,), (0,)), ((), ())),
                                 preferred_element_type=jnp.float32,
                                 precision=lax.Precision.HIGHEST)
        pp = jax.nn.relu(pooled)
        pp = jax.nn.relu(
            jnp.dot(pp, wf1_ref[...], preferred_element_type=jnp.float32)
            + bf1_ref[...])
        o_ref[...] = jnp.dot(pp, wf2_ref[...],
                             preferred_element_type=jnp.float32) + bf2_ref[...]

    return pl.pallas_call(
        body,
        out_shape=jax.ShapeDtypeStruct((_G, 1), jnp.float32),
    )(parts_a, parts_b, b5, g5, be5, batch2, Wfc1, bfc1, Wfc2, bfc2)


def kernel(x, edge_index, edge_weight, batch,
           W1, b1, g1, be1, W2, b2, g2, be2, W3, b3, g3, be3,
           W4, b4, g4, be4, W5, b5, g5, be5, Wfc1, bfc1, Wfc2, bfc2):
    loop = jnp.arange(_N, dtype=edge_index.dtype)
    pad = _EPAD2 - _ETOT
    padidx = (jnp.arange(pad, dtype=jnp.int32) * 37) % _N
    row = jnp.concatenate([edge_index[0], loop, padidx])
    col = jnp.concatenate([edge_index[1], loop, padidx])
    ew = jnp.concatenate([edge_weight, jnp.ones((_N,), edge_weight.dtype),
                          jnp.zeros((pad,), edge_weight.dtype)])
    rc = jnp.stack([row.reshape(_TOTCH, _CH), col.reshape(_TOTCH, _CH)],
                   axis=1)

    r2 = lambda a: a.reshape(1, -1)
    z16 = jnp.zeros((_NPAD, 16), jnp.float32)
    z32 = jnp.zeros((_NPAD, 32), jnp.float32)
    z64 = jnp.zeros((_NPAD, 64), jnp.float32)
    z128 = jnp.zeros((_NPAD, 128), jnp.float32)

    deg_parts = _make_degree()(col, ew, z16)
    xw1, dis16 = _tc_first(deg_parts, x, W1)
    norm = _make_norm()(row, col, ew, dis16)

    p1 = _make_propagate(32)(xw1, rc, norm, z32)
    (xw2,) = _tc_mid(p1, r2(b1), r2(g1), r2(be1), "relu_bn", W2)

    p2 = _make_propagate(64)(xw2, rc, norm, z64)
    (xw3,) = _tc_mid(p2, r2(b2), r2(g2), r2(be2), "relu_bn", W3)

    p3 = _make_propagate(128)(xw3, rc, norm, z128)
    (xw4,) = _tc_mid(p3, r2(b3), r2(g3), r2(be3), "relu_bn", W4)

    p4 = _make_propagate(128)(xw4, rc, norm, z128)
    xw5a, xw5b = _tc_mid(p4, r2(b4), r2(g4), r2(be4), "bn_relu",
                         W5[:, :128], W5[:, 128:])

    p5a = _make_propagate(128)(xw5a, rc, norm, z128)
    xw5b_seq = xw5b + 0.0 * p5a[0, :1, :1]
    p5b = _make_propagate(128)(xw5b_seq, rc, norm, z128)
    out = _tc_final(p5a, p5b, r2(b5), r2(g5), r2(be5),
                    batch.reshape(_N, 1), Wfc1, r2(bfc1), Wfc2, r2(bfc2))
    return out.reshape(-1)


# speedup vs baseline: 14.7453x; 1.4580x over previous
import functools

import jax
import jax.numpy as jnp
from jax import lax
from jax.experimental import pallas as pl
from jax.experimental.pallas import tpu as pltpu
from jax.experimental.pallas import tpu_sc as plsc

_N = 10000
_E = 320000
_G = 64
_NC = 2
_NS = 16
_NW = _NC * _NS
_CH = 128
_ETOT = _E + _N
_NCHUNK = 84
_EPAD = _NW * _CH * _NCHUNK
_EPAD2 = _EPAD + 12 * _CH
_KBY = {16: 3, 32: 3, 64: 3, 128: 1}
_TOTCH = _EPAD2 // _CH
_STRIPE = 632
_NPAD = _NS * _STRIPE

_SC_PARAMS = pltpu.CompilerParams(use_tc_tiling_on_sc=False)


@functools.lru_cache(maxsize=None)
def _mesh():
    return plsc.VectorSubcoreMesh(core_axis_name="c", subcore_axis_name="s")


def _wid_base():
    c = lax.axis_index("c")
    s = lax.axis_index("s")
    return c, s, (c * _NS + s) * _NCHUNK


@functools.lru_cache(maxsize=None)
def _make_degree():
  @functools.partial(
    pl.kernel,
    out_type=jax.ShapeDtypeStruct((_NC, _NPAD, 16), jnp.float32),
    mesh=_mesh(),
    compiler_params=_SC_PARAMS,
    scratch_types=[
        pltpu.VMEM((_CH,), jnp.int32),
        pltpu.VMEM((_CH,), jnp.float32),
        pltpu.VMEM((_CH, 16), jnp.float32),
        pltpu.VMEM_SHARED((_NPAD, 16), jnp.float32),
    ],
  )
  def _sc_degree(col_hbm, ew_hbm, zeros_hbm, out_hbm, colb, ewb, rows, acc):
    c, s, base = _wid_base()
    pltpu.sync_copy(zeros_hbm.at[pl.ds(s * _STRIPE, _STRIPE)],
                    acc.at[pl.ds(s * _STRIPE, _STRIPE)])
    plsc.subcore_barrier()

    def chunk_body(j, carry):
        off = (base + j) * _CH
        pltpu.sync_copy(col_hbm.at[pl.ds(off, _CH)], colb)
        pltpu.sync_copy(ew_hbm.at[pl.ds(off, _CH)], ewb)

        def splat(jj, ecarry):
            wv = ewb[pl.ds(jj * 16, 16)]
            for i in range(16):
                rows[jj * 16 + i, pl.ds(0, 16)] = jnp.full((16,), wv[i], jnp.float32)
            return ecarry

        lax.fori_loop(0, _CH // 16, splat, 0)
        pltpu.sync_copy(rows, acc.at[colb], add=True)
        return carry

    lax.fori_loop(0, _NCHUNK, chunk_body, 0)
    plsc.subcore_barrier()
    pltpu.sync_copy(acc.at[pl.ds(s * _STRIPE, _STRIPE)],
                    out_hbm.at[c, pl.ds(s * _STRIPE, _STRIPE)])

  return _sc_degree


@functools.lru_cache(maxsize=None)
def _make_norm():
  _K = 3
  _MAC = _NCHUNK // _K

  @functools.partial(
    pl.kernel,
    out_type=jax.ShapeDtypeStruct((_EPAD2, 16), jnp.float32),
    mesh=_mesh(),
    compiler_params=_SC_PARAMS,
    scratch_types=(
        [pltpu.VMEM((_K, 2, _CH), jnp.int32) for _ in range(4)]
        + [pltpu.VMEM((_K * _CH,), jnp.float32) for _ in range(4)]
        + [pltpu.VMEM((_K * _CH, 16), jnp.float32) for _ in range(4)]
        + [pltpu.SemaphoreType.DMA for _ in range(8)]
    ),
  )
  def _sc_norm(rc_hbm, ew_hbm, dis16_hbm, out_hbm,
               rc0, rc1, rc2, rc3, ew0, ew1, ew2, ew3,
               da0, da1, db0, db1,
               si0, si1, si2, si3, sg0, sg1, st0, st1):
    _, _, base = _wid_base()
    rc = (rc0, rc1, rc2, rc3)
    ew = (ew0, ew1, ew2, ew3)
    da = (da0, da1)
    db = (db0, db1)
    si = (si0, si1, si2, si3)
    sg = (sg0, sg1)
    st = (st0, st1)

    def idx_copies(m, b):
        ch0 = base + m * _K
        return (
            pltpu.make_async_copy(rc_hbm.at[pl.ds(ch0, _K)], rc[b], si[b]),
            pltpu.make_async_copy(ew_hbm.at[pl.ds(ch0 * _CH, _K * _CH)],
                                  ew[b], si[b]),
        )

    def gather_copies(b4, b2):
        cps = []
        for t in range(_K):
            cps.append(pltpu.make_async_copy(
                dis16_hbm.at[rc[b4].at[t, 0]],
                da[b2].at[pl.ds(t * _CH, _CH)], sg[b2]))
            cps.append(pltpu.make_async_copy(
                dis16_hbm.at[rc[b4].at[t, 1]],
                db[b2].at[pl.ds(t * _CH, _CH)], sg[b2]))
        return tuple(cps)

    def store_copy(m, b2):
        off = (base + m * _K) * _CH
        return pltpu.make_async_copy(da[b2],
                                     out_hbm.at[pl.ds(off, _K * _CH)],
                                     st[b2])

    def macro_step(g, r):
        m = 4 * g + r
        i4, i2 = r % 4, r % 2
        j4, j2 = (r + 1) % 4, (r + 1) % 2
        for cp in gather_copies(i4, i2):
            cp.wait()
        for cp in idx_copies(m + 1, j4):
            cp.wait()

        def drain_prev_store():
            store_copy(m - 1, j2).wait()

        if r == 0:
            @pl.when(m > 0)
            def _():
                drain_prev_store()
        else:
            drain_prev_store()

        for cp in idx_copies(m + 3, (r + 3) % 4):
            cp.start()
        for cp in gather_copies(j4, j2):
            cp.start()

        def emul(jj, ecarry):
            wv = ew[i4][pl.ds(jj * 16, 16)]
            for i in range(16):
                e = jj * 16 + i
                sl = pl.ds(0, 16)
                da[i2][e, sl] = (da[i2][e, sl] * wv[i]) * db[i2][e, sl]
            return ecarry

        lax.fori_loop(0, _K * _CH // 16, emul, 0)
        store_copy(m, i2).start()

    pltpu.sync_copy(rc_hbm.at[pl.ds(base, _K)], rc0)
    pltpu.sync_copy(ew_hbm.at[pl.ds(base * _CH, _K * _CH)], ew0)
    for cp in idx_copies(1, 1):
        cp.start()
    for cp in idx_copies(2, 2):
        cp.start()
    for cp in gather_copies(0, 0):
        cp.start()

    def group_body(g, carry):
        macro_step(g, 0)
        macro_step(g, 1)
        macro_step(g, 2)
        macro_step(g, 3)
        return carry

    lax.fori_loop(0, _MAC // 4, group_body, 0)

    store_copy(_MAC - 1, 1).wait()
    for cp in gather_copies(0, 0):
        cp.wait()
    for cp in idx_copies(0, 1):
        cp.wait()
    for cp in idx_copies(0, 2):
        cp.wait()

  return _sc_norm


@functools.lru_cache(maxsize=None)
def _make_propagate(D):
    _K = _KBY[D]
    _MAC = _NCHUNK // _K
    assert _MAC % 4 == 0

    @functools.partial(
        pl.kernel,
        out_type=jax.ShapeDtypeStruct((_NC, _NPAD, D), jnp.float32),
        mesh=_mesh(),
        compiler_params=_SC_PARAMS,
        scratch_types=(
            [pltpu.VMEM((_K, 2, _CH), jnp.int32) for _ in range(4)]
            + [pltpu.VMEM((_K * _CH, 16), jnp.float32) for _ in range(4)]
            + [pltpu.VMEM((_K * _CH, D), jnp.float32) for _ in range(2)]
            + [pltpu.VMEM_SHARED((_NPAD, D), jnp.float32)]
            + [pltpu.SemaphoreType.DMA for _ in range(8)]
        ),
    )
    def prop(src_hbm, rc_hbm, norm_hbm, zeros_hbm, out_hbm,
             rc0, rc1, rc2, rc3, nb0, nb1, nb2, nb3, rows0, rows1, acc,
             si0, si1, si2, si3, sg0, sg1, ss0, ss1):
        c, s, base = _wid_base()
        rc = (rc0, rc1, rc2, rc3)
        nb = (nb0, nb1, nb2, nb3)
        rows = (rows0, rows1)
        si = (si0, si1, si2, si3)
        sg = (sg0, sg1)
        ss = (ss0, ss1)

        pltpu.sync_copy(zeros_hbm.at[pl.ds(s * _STRIPE, _STRIPE)],
                        acc.at[pl.ds(s * _STRIPE, _STRIPE)])
        plsc.subcore_barrier()

        def idx_copies(m, b):
            ch0 = base + m * _K
            return (
                pltpu.make_async_copy(rc_hbm.at[pl.ds(ch0, _K)], rc[b], si[b]),
                pltpu.make_async_copy(norm_hbm.at[pl.ds(ch0 * _CH, _K * _CH)],
                                      nb[b], si[b]),
            )

        def gather_copies(b4, b2):
            return tuple(
                pltpu.make_async_copy(src_hbm.at[rc[b4].at[t, 0]],
                                      rows[b2].at[pl.ds(t * _CH, _CH)],
                                      sg[b2])
                for t in range(_K))

        def scatter_copies(b4, b2):
            return tuple(
                pltpu.make_async_copy(rows[b2].at[pl.ds(t * _CH, _CH)],
                                      acc.at[rc[b4].at[t, 1]], ss[b2])
                for t in range(_K))

        def macro_step(g, r):
            m = 4 * g + r
            i4, i2 = r % 4, r % 2
            j4, j2 = (r + 1) % 4, (r + 1) % 2
            for cp in gather_copies(i4, i2):
                cp.wait()
            for cp in idx_copies(m + 1, j4):
                cp.wait()

            def drain_prev_scatter():
                for cp in scatter_copies((r - 1) % 4, j2):
                    cp.wait()

            if r == 0:
                @pl.when(m > 0)
                def _():
                    drain_prev_scatter()
            else:
                drain_prev_scatter()

            for cp in idx_copies(m + 3, (r + 3) % 4):
                cp.start()
            for cp in gather_copies(j4, j2):
                cp.start()

            def escale(e2, ecarry):
                for u in range(2):
                    e = e2 * 2 + u
                    nv = nb[i4][e, pl.ds(0, 16)]
                    for k in range(D // 16):
                        sl = pl.ds(k * 16, 16)
                        rows[i2][e, sl] = rows[i2][e, sl] * nv
                return ecarry

            lax.fori_loop(0, _K * _CH // 2, escale, 0)
            for t in range(_K):
                pltpu.async_copy(rows[i2].at[pl.ds(t * _CH, _CH)],
                                 acc.at[rc[i4].at[t, 1]], ss[i2], add=True)

        pltpu.sync_copy(rc_hbm.at[pl.ds(base, _K)], rc0)
        pltpu.sync_copy(norm_hbm.at[pl.ds(base * _CH, _K * _CH)], nb0)
        for cp in idx_copies(1, 1):
            cp.start()
        for cp in idx_copies(2, 2):
            cp.start()
        for cp in gather_copies(0, 0):
            cp.start()

        def group_body(g, carry):
            macro_step(g, 0)
            macro_step(g, 1)
            macro_step(g, 2)
            macro_step(g, 3)
            return carry

        lax.fori_loop(0, _MAC // 4, group_body, 0)

        for cp in scatter_copies(3, 1):
            cp.wait()
        for cp in gather_copies(0, 0):
            cp.wait()
        for cp in idx_copies(0, 1):
            cp.wait()
        for cp in idx_copies(0, 2):
            cp.wait()

        plsc.subcore_barrier()
        pltpu.sync_copy(acc.at[pl.ds(s * _STRIPE, _STRIPE)],
                        out_hbm.at[c, pl.ds(s * _STRIPE, _STRIPE)])

    return prop


def _bn_in_kernel(u, g, be):
    m = jnp.mean(u, axis=0, keepdims=True)
    v = jnp.mean((u - m) ** 2, axis=0, keepdims=True)
    return (u - m) * lax.rsqrt(v + 1e-5) * g + be


def _tc_first(deg_parts, x, W1):
    def body(dp_ref, x_ref, w_ref, xw_ref, dis_ref):
        dp = dp_ref[...]
        deg = dp[0, :_N] + dp[1, :_N]
        dis_ref[...] = jnp.where(deg > 0, lax.rsqrt(deg), 0.0)
        xw_ref[...] = jnp.dot(x_ref[...], w_ref[...],
                              preferred_element_type=jnp.float32)

    return pl.pallas_call(
        body,
        out_shape=[jax.ShapeDtypeStruct((_N, 32), jnp.float32),
                   jax.ShapeDtypeStruct((_N, 16), jnp.float32)],
    )(deg_parts, x, W1)


def _tc_mid(parts, b, g, be, mode, *Ws):
    def body(*refs):
        p_ref, b_ref, g_ref, be_ref = refs[:4]
        w_refs = refs[4:4 + len(Ws)]
        o_refs = refs[4 + len(Ws):]
        p = p_ref[...]
        t = (p[0, :_N] + p[1, :_N]) + b_ref[...]
        if mode == "relu_bn":
            h = _bn_in_kernel(jax.nn.relu(t), g_ref[...], be_ref[...])
        else:
            h = jax.nn.relu(_bn_in_kernel(t, g_ref[...], be_ref[...]))
        for w_ref, o_ref in zip(w_refs, o_refs):
            o_ref[...] = jnp.dot(h, w_ref[...],
                                 preferred_element_type=jnp.float32)

    return pl.pallas_call(
        body,
        out_shape=[jax.ShapeDtypeStruct((_N, W.shape[1]), jnp.float32)
                   for W in Ws],
    )(parts, b, g, be, *Ws)


def _tc_final(parts_a, parts_b, b5, g5, be5, batch2, Wfc1, bfc1, Wfc2, bfc2):
    def body(pa_ref, pb_ref, b_ref, g_ref, be_ref, bt_ref,
             wf1_ref, bf1_ref, wf2_ref, bf2_ref, o_ref):
        pa = pa_ref[...]
        pb = pb_ref[...]
        t = jnp.concatenate([pa[0, :_N] + pa[1, :_N],
                             pb[0, :_N] + pb[1, :_N]], axis=1) + b_ref[...]
        h = _bn_in_kernel(t, g_ref[...], be_ref[...])
        oh = (bt_ref[...] == lax.broadcasted_iota(jnp.int32, (1, _G), 1))
        oh = oh.astype(jnp.float32)
        pooled = lax.dot_general(oh, h, (((0,), (0,)), ((), ())),
                                 preferred_element_type=jnp.float32,
                                 precision=lax.Precision.HIGHEST)
        pp = jax.nn.relu(pooled)
        pp = jax.nn.relu(
            jnp.dot(pp, wf1_ref[...], preferred_element_type=jnp.float32)
            + bf1_ref[...])
        o_ref[...] = jnp.dot(pp, wf2_ref[...],
                             preferred_element_type=jnp.float32) + bf2_ref[...]

    return pl.pallas_call(
        body,
        out_shape=jax.ShapeDtypeStruct((_G, 1), jnp.float32),
    )(parts_a, parts_b, b5, g5, be5, batch2, Wfc1, bfc1, Wfc2, bfc2)


def kernel(x, edge_index, edge_weight, batch,
           W1, b1, g1, be1, W2, b2, g2, be2, W3, b3, g3, be3,
           W4, b4, g4, be4, W5, b5, g5, be5, Wfc1, bfc1, Wfc2, bfc2):
    loop = jnp.arange(_N, dtype=edge_index.dtype)
    pad = _EPAD2 - _ETOT
    padidx = (jnp.arange(pad, dtype=jnp.int32) * 37) % _N
    row = jnp.concatenate([edge_index[0], loop, padidx])
    col = jnp.concatenate([edge_index[1], loop, padidx])
    ew = jnp.concatenate([edge_weight, jnp.ones((_N,), edge_weight.dtype),
                          jnp.zeros((pad,), edge_weight.dtype)])
    rc = jnp.stack([row.reshape(_TOTCH, _CH), col.reshape(_TOTCH, _CH)],
                   axis=1)

    r2 = lambda a: a.reshape(1, -1)
    z16 = jnp.zeros((_NPAD, 16), jnp.float32)
    z32 = jnp.zeros((_NPAD, 32), jnp.float32)
    z64 = jnp.zeros((_NPAD, 64), jnp.float32)
    z128 = jnp.zeros((_NPAD, 128), jnp.float32)

    deg_parts = _make_degree()(col, ew, z16)
    xw1, dis16 = _tc_first(deg_parts, x, W1)
    norm = _make_norm()(rc, ew, dis16)

    p1 = _make_propagate(32)(xw1, rc, norm, z32)
    (xw2,) = _tc_mid(p1, r2(b1), r2(g1), r2(be1), "relu_bn", W2)

    p2 = _make_propagate(64)(xw2, rc, norm, z64)
    (xw3,) = _tc_mid(p2, r2(b2), r2(g2), r2(be2), "relu_bn", W3)

    p3 = _make_propagate(128)(xw3, rc, norm, z128)
    (xw4,) = _tc_mid(p3, r2(b3), r2(g3), r2(be3), "relu_bn", W4)

    p4 = _make_propagate(128)(xw4, rc, norm, z128)
    xw5a, xw5b = _tc_mid(p4, r2(b4), r2(g4), r2(be4), "bn_relu",
                         W5[:, :128], W5[:, 128:])

    p5a = _make_propagate(128)(xw5a, rc, norm, z128)
    xw5b_seq = xw5b + 0.0 * p5a[0, :1, :1]
    p5b = _make_propagate(128)(xw5b_seq, rc, norm, z128)
    out = _tc_final(p5a, p5b, r2(b5), r2(g5), r2(be5),
                    batch.reshape(_N, 1), Wfc1, r2(bfc1), Wfc2, r2(bfc2))
    return out.reshape(-1)


# speedup vs baseline: 15.7100x; 1.0654x over previous
import functools

import jax
import jax.numpy as jnp
from jax import lax
from jax.experimental import pallas as pl
from jax.experimental.pallas import tpu as pltpu
from jax.experimental.pallas import tpu_sc as plsc

_N = 10000
_E = 320000
_G = 64
_NC = 2
_NS = 16
_NW = _NC * _NS
_CH = 128
_ETOT = _E + _N
_NCHUNK = 84
_EPAD = _NW * _CH * _NCHUNK
_EPAD2 = _EPAD + 12 * _CH
_KBY = {16: 3, 32: 3, 64: 3, 128: 1}
_TOTCH = _EPAD2 // _CH
_STRIPE = 632
_NPAD = _NS * _STRIPE

_SC_PARAMS = pltpu.CompilerParams(use_tc_tiling_on_sc=False)


@functools.lru_cache(maxsize=None)
def _mesh():
    return plsc.VectorSubcoreMesh(core_axis_name="c", subcore_axis_name="s")


def _wid_base():
    c = lax.axis_index("c")
    s = lax.axis_index("s")
    return c, s, (c * _NS + s) * _NCHUNK


@functools.lru_cache(maxsize=None)
def _make_degree():
  _K = 3
  _MAC = _NCHUNK // _K

  @functools.partial(
    pl.kernel,
    out_type=jax.ShapeDtypeStruct((_NC, _NPAD, 16), jnp.float32),
    mesh=_mesh(),
    compiler_params=_SC_PARAMS,
    scratch_types=(
        [pltpu.VMEM((_K, 2, _CH), jnp.int32) for _ in range(4)]
        + [pltpu.VMEM((_K * _CH,), jnp.float32) for _ in range(4)]
        + [pltpu.VMEM((_K * _CH, 16), jnp.float32) for _ in range(2)]
        + [pltpu.VMEM_SHARED((_NPAD, 16), jnp.float32)]
        + [pltpu.SemaphoreType.DMA for _ in range(6)]
    ),
  )
  def _sc_degree(rc_hbm, ew_hbm, zeros_hbm, out_hbm,
                 rc0, rc1, rc2, rc3, ew0, ew1, ew2, ew3, rows0, rows1, acc,
                 si0, si1, si2, si3, ss0, ss1):
    c, s, base = _wid_base()
    rc = (rc0, rc1, rc2, rc3)
    ew = (ew0, ew1, ew2, ew3)
    rows = (rows0, rows1)
    si = (si0, si1, si2, si3)
    ss = (ss0, ss1)

    pltpu.sync_copy(zeros_hbm.at[pl.ds(s * _STRIPE, _STRIPE)],
                    acc.at[pl.ds(s * _STRIPE, _STRIPE)])
    plsc.subcore_barrier()

    def idx_copies(m, b):
        ch0 = base + m * _K
        return (
            pltpu.make_async_copy(rc_hbm.at[pl.ds(ch0, _K)], rc[b], si[b]),
            pltpu.make_async_copy(ew_hbm.at[pl.ds(ch0 * _CH, _K * _CH)],
                                  ew[b], si[b]),
        )

    def scatter_copies(b4, b2):
        return tuple(
            pltpu.make_async_copy(rows[b2].at[pl.ds(t * _CH, _CH)],
                                  acc.at[rc[b4].at[t, 1]], ss[b2])
            for t in range(_K))

    def macro_step(g, r):
        m = 4 * g + r
        i4, i2 = r % 4, r % 2
        for cp in idx_copies(m, i4):
            cp.wait()

        def drain_prev_scatter():
            for cp in scatter_copies((r - 1) % 4, (r + 1) % 2):
                cp.wait()

        if r == 0:
            @pl.when(m > 0)
            def _():
                drain_prev_scatter()
        else:
            drain_prev_scatter()

        for cp in idx_copies(m + 3, (r + 3) % 4):
            cp.start()

        def splat(jj, ecarry):
            wv = ew[i4][pl.ds(jj * 16, 16)]
            for i in range(16):
                rows[i2][jj * 16 + i, pl.ds(0, 16)] = (
                    jnp.full((16,), wv[i], jnp.float32))
            return ecarry

        lax.fori_loop(0, _K * _CH // 16, splat, 0)
        for t in range(_K):
            pltpu.async_copy(rows[i2].at[pl.ds(t * _CH, _CH)],
                             acc.at[rc[i4].at[t, 1]], ss[i2], add=True)

    for cp in idx_copies(0, 0):
        cp.start()
    for cp in idx_copies(1, 1):
        cp.start()
    for cp in idx_copies(2, 2):
        cp.start()

    def group_body(g, carry):
        macro_step(g, 0)
        macro_step(g, 1)
        macro_step(g, 2)
        macro_step(g, 3)
        return carry

    lax.fori_loop(0, _MAC // 4, group_body, 0)

    for cp in scatter_copies(3, 1):
        cp.wait()
    for cp in idx_copies(0, 0):
        cp.wait()
    for cp in idx_copies(0, 1):
        cp.wait()
    for cp in idx_copies(0, 2):
        cp.wait()

    plsc.subcore_barrier()
    pltpu.sync_copy(acc.at[pl.ds(s * _STRIPE, _STRIPE)],
                    out_hbm.at[c, pl.ds(s * _STRIPE, _STRIPE)])

  return _sc_degree


@functools.lru_cache(maxsize=None)
def _make_norm():
  _K = 3
  _MAC = _NCHUNK // _K

  @functools.partial(
    pl.kernel,
    out_type=jax.ShapeDtypeStruct((_EPAD2, 16), jnp.float32),
    mesh=_mesh(),
    compiler_params=_SC_PARAMS,
    scratch_types=(
        [pltpu.VMEM((_K, 2, _CH), jnp.int32) for _ in range(4)]
        + [pltpu.VMEM((_K * _CH,), jnp.float32) for _ in range(4)]
        + [pltpu.VMEM((_K * _CH, 16), jnp.float32) for _ in range(4)]
        + [pltpu.SemaphoreType.DMA for _ in range(8)]
    ),
  )
  def _sc_norm(rc_hbm, ew_hbm, dis16_hbm, out_hbm,
               rc0, rc1, rc2, rc3, ew0, ew1, ew2, ew3,
               da0, da1, db0, db1,
               si0, si1, si2, si3, sg0, sg1, st0, st1):
    _, _, base = _wid_base()
    rc = (rc0, rc1, rc2, rc3)
    ew = (ew0, ew1, ew2, ew3)
    da = (da0, da1)
    db = (db0, db1)
    si = (si0, si1, si2, si3)
    sg = (sg0, sg1)
    st = (st0, st1)

    def idx_copies(m, b):
        ch0 = base + m * _K
        return (
            pltpu.make_async_copy(rc_hbm.at[pl.ds(ch0, _K)], rc[b], si[b]),
            pltpu.make_async_copy(ew_hbm.at[pl.ds(ch0 * _CH, _K * _CH)],
                                  ew[b], si[b]),
        )

    def gather_copies(b4, b2):
        cps = []
        for t in range(_K):
            cps.append(pltpu.make_async_copy(
                dis16_hbm.at[rc[b4].at[t, 0]],
                da[b2].at[pl.ds(t * _CH, _CH)], sg[b2]))
            cps.append(pltpu.make_async_copy(
                dis16_hbm.at[rc[b4].at[t, 1]],
                db[b2].at[pl.ds(t * _CH, _CH)], sg[b2]))
        return tuple(cps)

    def store_copy(m, b2):
        off = (base + m * _K) * _CH
        return pltpu.make_async_copy(da[b2],
                                     out_hbm.at[pl.ds(off, _K * _CH)],
                                     st[b2])

    def macro_step(g, r):
        m = 4 * g + r
        i4, i2 = r % 4, r % 2
        j4, j2 = (r + 1) % 4, (r + 1) % 2
        for cp in gather_copies(i4, i2):
            cp.wait()
        for cp in idx_copies(m + 1, j4):
            cp.wait()

        def drain_prev_store():
            store_copy(m - 1, j2).wait()

        if r == 0:
            @pl.when(m > 0)
            def _():
                drain_prev_store()
        else:
            drain_prev_store()

        for cp in idx_copies(m + 3, (r + 3) % 4):
            cp.start()
        for cp in gather_copies(j4, j2):
            cp.start()

        def emul(jj, ecarry):
            wv = ew[i4][pl.ds(jj * 16, 16)]
            for i in range(16):
                e = jj * 16 + i
                sl = pl.ds(0, 16)
                da[i2][e, sl] = (da[i2][e, sl] * wv[i]) * db[i2][e, sl]
            return ecarry

        lax.fori_loop(0, _K * _CH // 16, emul, 0)
        store_copy(m, i2).start()

    pltpu.sync_copy(rc_hbm.at[pl.ds(base, _K)], rc0)
    pltpu.sync_copy(ew_hbm.at[pl.ds(base * _CH, _K * _CH)], ew0)
    for cp in idx_copies(1, 1):
        cp.start()
    for cp in idx_copies(2, 2):
        cp.start()
    for cp in gather_copies(0, 0):
        cp.start()

    def group_body(g, carry):
        macro_step(g, 0)
        macro_step(g, 1)
        macro_step(g, 2)
        macro_step(g, 3)
        return carry

    lax.fori_loop(0, _MAC // 4, group_body, 0)

    store_copy(_MAC - 1, 1).wait()
    for cp in gather_copies(0, 0):
        cp.wait()
    for cp in idx_copies(0, 1):
        cp.wait()
    for cp in idx_copies(0, 2):
        cp.wait()

  return _sc_norm


@functools.lru_cache(maxsize=None)
def _make_propagate(D):
    _K = _KBY[D]
    _MAC = _NCHUNK // _K
    assert _MAC % 4 == 0

    @functools.partial(
        pl.kernel,
        out_type=jax.ShapeDtypeStruct((_NC, _NPAD, D), jnp.float32),
        mesh=_mesh(),
        compiler_params=_SC_PARAMS,
        scratch_types=(
            [pltpu.VMEM((_K, 2, _CH), jnp.int32) for _ in range(4)]
            + [pltpu.VMEM((_K * _CH, 16), jnp.float32) for _ in range(4)]
            + [pltpu.VMEM((_K * _CH, D), jnp.float32) for _ in range(2)]
            + [pltpu.VMEM_SHARED((_NPAD, D), jnp.float32)]
            + [pltpu.SemaphoreType.DMA for _ in range(8)]
        ),
    )
    def prop(src_hbm, rc_hbm, norm_hbm, zeros_hbm, out_hbm,
             rc0, rc1, rc2, rc3, nb0, nb1, nb2, nb3, rows0, rows1, acc,
             si0, si1, si2, si3, sg0, sg1, ss0, ss1):
        c, s, base = _wid_base()
        rc = (rc0, rc1, rc2, rc3)
        nb = (nb0, nb1, nb2, nb3)
        rows = (rows0, rows1)
        si = (si0, si1, si2, si3)
        sg = (sg0, sg1)
        ss = (ss0, ss1)

        pltpu.sync_copy(zeros_hbm.at[pl.ds(s * _STRIPE, _STRIPE)],
                        acc.at[pl.ds(s * _STRIPE, _STRIPE)])
        plsc.subcore_barrier()

        def idx_copies(m, b):
            ch0 = base + m * _K
            return (
                pltpu.make_async_copy(rc_hbm.at[pl.ds(ch0, _K)], rc[b], si[b]),
                pltpu.make_async_copy(norm_hbm.at[pl.ds(ch0 * _CH, _K * _CH)],
                                      nb[b], si[b]),
            )

        def gather_copies(b4, b2):
            return tuple(
                pltpu.make_async_copy(src_hbm.at[rc[b4].at[t, 0]],
                                      rows[b2].at[pl.ds(t * _CH, _CH)],
                                      sg[b2])
                for t in range(_K))

        def scatter_copies(b4, b2):
            return tuple(
                pltpu.make_async_copy(rows[b2].at[pl.ds(t * _CH, _CH)],
                                      acc.at[rc[b4].at[t, 1]], ss[b2])
                for t in range(_K))

        def macro_step(g, r):
            m = 4 * g + r
            i4, i2 = r % 4, r % 2
            j4, j2 = (r + 1) % 4, (r + 1) % 2
            for cp in gather_copies(i4, i2):
                cp.wait()
            for cp in idx_copies(m + 1, j4):
                cp.wait()

            def drain_prev_scatter():
                for cp in scatter_copies((r - 1) % 4, j2):
                    cp.wait()

            if r == 0:
                @pl.when(m > 0)
                def _():
                    drain_prev_scatter()
            else:
                drain_prev_scatter()

            for cp in idx_copies(m + 3, (r + 3) % 4):
                cp.start()
            for cp in gather_copies(j4, j2):
                cp.start()

            def escale(e2, ecarry):
                for u in range(2):
                    e = e2 * 2 + u
                    nv = nb[i4][e, pl.ds(0, 16)]
                    for k in range(D // 16):
                        sl = pl.ds(k * 16, 16)
                        rows[i2][e, sl] = rows[i2][e, sl] * nv
                return ecarry

            lax.fori_loop(0, _K * _CH // 2, escale, 0)
            for t in range(_K):
                pltpu.async_copy(rows[i2].at[pl.ds(t * _CH, _CH)],
                                 acc.at[rc[i4].at[t, 1]], ss[i2], add=True)

        pltpu.sync_copy(rc_hbm.at[pl.ds(base, _K)], rc0)
        pltpu.sync_copy(norm_hbm.at[pl.ds(base * _CH, _K * _CH)], nb0)
        for cp in idx_copies(1, 1):
            cp.start()
        for cp in idx_copies(2, 2):
            cp.start()
        for cp in gather_copies(0, 0):
            cp.start()

        def group_body(g, carry):
            macro_step(g, 0)
            macro_step(g, 1)
            macro_step(g, 2)
            macro_step(g, 3)
            return carry

        lax.fori_loop(0, _MAC // 4, group_body, 0)

        for cp in scatter_copies(3, 1):
            cp.wait()
        for cp in gather_copies(0, 0):
            cp.wait()
        for cp in idx_copies(0, 1):
            cp.wait()
        for cp in idx_copies(0, 2):
            cp.wait()

        plsc.subcore_barrier()
        pltpu.sync_copy(acc.at[pl.ds(s * _STRIPE, _STRIPE)],
                        out_hbm.at[c, pl.ds(s * _STRIPE, _STRIPE)])

    return prop


def _bn_in_kernel(u, g, be):
    m = jnp.mean(u, axis=0, keepdims=True)
    v = jnp.mean((u - m) ** 2, axis=0, keepdims=True)
    return (u - m) * lax.rsqrt(v + 1e-5) * g + be


def _tc_first(deg_parts, x, W1):
    def body(dp_ref, x_ref, w_ref, xw_ref, dis_ref):
        dp = dp_ref[...]
        deg = dp[0, :_N] + dp[1, :_N]
        dis_ref[...] = jnp.where(deg > 0, lax.rsqrt(deg), 0.0)
        xw_ref[...] = jnp.dot(x_ref[...], w_ref[...],
                              preferred_element_type=jnp.float32)

    return pl.pallas_call(
        body,
        out_shape=[jax.ShapeDtypeStruct((_N, 32), jnp.float32),
                   jax.ShapeDtypeStruct((_N, 16), jnp.float32)],
    )(deg_parts, x, W1)


def _tc_mid(parts, b, g, be, mode, *Ws):
    def body(*refs):
        p_ref, b_ref, g_ref, be_ref = refs[:4]
        w_refs = refs[4:4 + len(Ws)]
        o_refs = refs[4 + len(Ws):]
        p = p_ref[...]
        t = (p[0, :_N] + p[1, :_N]) + b_ref[...]
        if mode == "relu_bn":
            h = _bn_in_kernel(jax.nn.relu(t), g_ref[...], be_ref[...])
        else:
            h = jax.nn.relu(_bn_in_kernel(t, g_ref[...], be_ref[...]))
        for w_ref, o_ref in zip(w_refs, o_refs):
            o_ref[...] = jnp.dot(h, w_ref[...],
                                 preferred_element_type=jnp.float32)

    return pl.pallas_call(
        body,
        out_shape=[jax.ShapeDtypeStruct((_N, W.shape[1]), jnp.float32)
                   for W in Ws],
    )(parts, b, g, be, *Ws)


def _tc_final(parts_a, parts_b, b5, g5, be5, batch2, Wfc1, bfc1, Wfc2, bfc2):
    def body(pa_ref, pb_ref, b_ref, g_ref, be_ref, bt_ref,
             wf1_ref, bf1_ref, wf2_ref, bf2_ref, o_ref):
        pa = pa_ref[...]
        pb = pb_ref[...]
        t = jnp.concatenate([pa[0, :_N] + pa[1, :_N],
                             pb[0, :_N] + pb[1, :_N]], axis=1) + b_ref[...]
        h = _bn_in_kernel(t, g_ref[...], be_ref[...])
        oh = (bt_ref[...] == lax.broadcasted_iota(jnp.int32, (1, _G), 1))
        oh = oh.astype(jnp.float32)
        pooled = lax.dot_general(oh, h, (((0,), (0,)), ((), ())),
                                 preferred_element_type=jnp.float32,
                                 precision=lax.Precision.HIGHEST)
        pp = jax.nn.relu(pooled)
        pp = jax.nn.relu(
            jnp.dot(pp, wf1_ref[...], preferred_element_type=jnp.float32)
            + bf1_ref[...])
        o_ref[...] = jnp.dot(pp, wf2_ref[...],
                             preferred_element_type=jnp.float32) + bf2_ref[...]

    return pl.pallas_call(
        body,
        out_shape=jax.ShapeDtypeStruct((_G, 1), jnp.float32),
    )(parts_a, parts_b, b5, g5, be5, batch2, Wfc1, bfc1, Wfc2, bfc2)


def kernel(x, edge_index, edge_weight, batch,
           W1, b1, g1, be1, W2, b2, g2, be2, W3, b3, g3, be3,
           W4, b4, g4, be4, W5, b5, g5, be5, Wfc1, bfc1, Wfc2, bfc2):
    loop = jnp.arange(_N, dtype=edge_index.dtype)
    pad = _EPAD2 - _ETOT
    padidx = (jnp.arange(pad, dtype=jnp.int32) * 37) % _N
    row = jnp.concatenate([edge_index[0], loop, padidx])
    col = jnp.concatenate([edge_index[1], loop, padidx])
    ew = jnp.concatenate([edge_weight, jnp.ones((_N,), edge_weight.dtype),
                          jnp.zeros((pad,), edge_weight.dtype)])
    rc = jnp.stack([row.reshape(_TOTCH, _CH), col.reshape(_TOTCH, _CH)],
                   axis=1)

    r2 = lambda a: a.reshape(1, -1)
    z16 = jnp.zeros((_NPAD, 16), jnp.float32)
    z32 = jnp.zeros((_NPAD, 32), jnp.float32)
    z64 = jnp.zeros((_NPAD, 64), jnp.float32)
    z128 = jnp.zeros((_NPAD, 128), jnp.float32)

    deg_parts = _make_degree()(rc, ew, z16)
    xw1, dis16 = _tc_first(deg_parts, x, W1)
    norm = _make_norm()(rc, ew, dis16)

    p1 = _make_propagate(32)(xw1, rc, norm, z32)
    (xw2,) = _tc_mid(p1, r2(b1), r2(g1), r2(be1), "relu_bn", W2)

    p2 = _make_propagate(64)(xw2, rc, norm, z64)
    (xw3,) = _tc_mid(p2, r2(b2), r2(g2), r2(be2), "relu_bn", W3)

    p3 = _make_propagate(128)(xw3, rc, norm, z128)
    (xw4,) = _tc_mid(p3, r2(b3), r2(g3), r2(be3), "relu_bn", W4)

    p4 = _make_propagate(128)(xw4, rc, norm, z128)
    xw5a, xw5b = _tc_mid(p4, r2(b4), r2(g4), r2(be4), "bn_relu",
                         W5[:, :128], W5[:, 128:])

    p5a = _make_propagate(128)(xw5a, rc, norm, z128)
    xw5b_seq = xw5b + 0.0 * p5a[0, :1, :1]
    p5b = _make_propagate(128)(xw5b_seq, rc, norm, z128)
    out = _tc_final(p5a, p5b, r2(b5), r2(g5), r2(be5),
                    batch.reshape(_N, 1), Wfc1, r2(bfc1), Wfc2, r2(bfc2))
    return out.reshape(-1)
